# Initial kernel scaffold; baseline (speedup 1.0000x reference)
import jax
import jax.numpy as jnp
from jax.experimental import pallas as pl


def kernel(x, edge_index, params):
    raise NotImplementedError("write your pallas kernel here")


# speedup vs baseline: 7.4876x; 7.4876x over previous
import functools

import jax
import jax.numpy as jnp
from jax import lax
from jax.experimental import pallas as pl
from jax.experimental.pallas import tpu as pltpu
from jax.experimental.pallas import tpu_sc as plsc

NN = 10000
NPAD = 10240
EE = 320000
HID = 256
HALF = 128
MROW = 384
NW = 32
EPW = EE // NW
CHUNK = 40
NCHUNKS = EPW // CHUNK
EB = 512
NSL = 2000

_PREC = jax.lax.Precision.HIGHEST
_F32 = jnp.float32


def _proj_body(x_ref, w_ref, was_ref, wad_ref, h_ref, as_ref, ad_ref):
    h_ref[...] = jnp.dot(x_ref[...], w_ref[...],
                         preferred_element_type=_F32, precision=_PREC)
    as_ref[...] = jnp.dot(x_ref[...], was_ref[...],
                          preferred_element_type=_F32, precision=_PREC)
    ad_ref[...] = jnp.dot(x_ref[...], wad_ref[...],
                          preferred_element_type=_F32, precision=_PREC)


def _proj(x, w, was16, wad16, rows_blk):
    d_in = x.shape[-1]
    ni = NN // rows_blk
    return pl.pallas_call(
        _proj_body,
        grid=(ni,),
        in_specs=[
            pl.BlockSpec((rows_blk, d_in), lambda i: (i, 0)),
            pl.BlockSpec((d_in, HID), lambda i: (0, 0)),
            pl.BlockSpec((d_in, HALF), lambda i: (0, 0)),
            pl.BlockSpec((d_in, HALF), lambda i: (0, 0)),
        ],
        out_specs=[
            pl.BlockSpec((rows_blk, HID), lambda i: (i, 0)),
            pl.BlockSpec((rows_blk, HALF), lambda i: (i, 0)),
            pl.BlockSpec((rows_blk, HALF), lambda i: (i, 0)),
        ],
        out_shape=[
            jax.ShapeDtypeStruct((NN, HID), _F32),
            jax.ShapeDtypeStruct((NPAD, HALF), _F32),
            jax.ShapeDtypeStruct((NPAD, HALF), _F32),
        ],
    )(x, w, was16, wad16)


def _edges_body(src_h, dst_h, as_h, ad_h, h_h, msg_h,
                src_v, dst_v, as_v, ad_v, h_v, m_v, sem):
    c = lax.axis_index("c")
    t = lax.axis_index("s")
    wbase = (c * 16 + t) * EPW
    head_idx = [jnp.full((16,), v // 2, jnp.int32) for v in range(HID // 16)]

    def _zpad(j, carry):
        for v in range((MROW - (HID + 16)) // 16):
            m_v[j, pl.ds(HID + 16 + v * 16, 16)] = jnp.zeros((16,), _F32)
        return carry

    lax.fori_loop(0, CHUNK, _zpad, 0)

    def _chunk_body(i, carry):
        base = wbase + i * CHUNK
        pltpu.sync_copy(src_h.at[pl.ds(base, CHUNK)], src_v)
        pltpu.sync_copy(dst_h.at[pl.ds(base, CHUNK)], dst_v)
        pltpu.async_copy(as_h.at[src_v], as_v, sem).wait()
        pltpu.async_copy(ad_h.at[dst_v], ad_v, sem).wait()
        pltpu.async_copy(h_h.at[src_v], h_v, sem).wait()

        def _edge_body(j, carry2):
            e = as_v[j, pl.ds(0, 16)] + ad_v[j, pl.ds(0, 16)]
            e = jnp.where(e >= 0.0, e, 0.2 * e)
            srow = jnp.exp(e)
            for v in range(HID // 16):
                m = srow.at[head_idx[v]].get(mode="promise_in_bounds")
                m_v[j, pl.ds(v * 16, 16)] = h_v[j, pl.ds(v * 16, 16)] * m
            m_v[j, pl.ds(HID, 16)] = srow
            return carry2

        lax.fori_loop(0, CHUNK, _edge_body, 0)
        pltpu.sync_copy(m_v, msg_h.at[pl.ds(base, CHUNK)])
        return carry

    lax.fori_loop(0, NCHUNKS, _chunk_body, 0)


_edges = pl.kernel(
    _edges_body,
    mesh=plsc.VectorSubcoreMesh(core_axis_name="c", subcore_axis_name="s"),
    out_type=[
        jax.ShapeDtypeStruct((EE, MROW), _F32),
    ],
    scratch_types=[
        pltpu.VMEM((CHUNK,), jnp.int32),
        pltpu.VMEM((CHUNK,), jnp.int32),
        pltpu.VMEM((CHUNK, HALF), _F32),
        pltpu.VMEM((CHUNK, HALF), _F32),
        pltpu.VMEM((CHUNK, HID), _F32),
        pltpu.VMEM((CHUNK, MROW), _F32),
        pltpu.SemaphoreType.DMA,
    ],
)


def _reduce_body(dst_ref, msg_ref, out_ref):
    k = pl.program_id(0)

    @pl.when(k == 0)
    def _():
        out_ref[...] = jnp.zeros_like(out_ref)

    dstv = dst_ref[0, :]
    msgb = msg_ref[...].astype(jnp.bfloat16)
    for ns in range(NN // NSL):
        ids = lax.broadcasted_iota(jnp.int32, (NSL, EB), 0) + ns * NSL
        p = (ids == dstv[None, :]).astype(jnp.bfloat16)
        mm = jnp.dot(p, msgb, preferred_element_type=_F32)
        out_ref[pl.ds(ns * NSL, NSL), :] += mm


def _reduce(dst2d, msg):
    return pl.pallas_call(
        _reduce_body,
        grid=(EE // EB,),
        in_specs=[
            pl.BlockSpec((1, EB), lambda k: (0, k)),
            pl.BlockSpec((EB, MROW), lambda k: (k, 0)),
        ],
        out_specs=pl.BlockSpec((NN, MROW), lambda k: (0, 0)),
        out_shape=jax.ShapeDtypeStruct((NN, MROW), _F32),
    )(dst2d, msg)


def _make_ffn_body(has_res):
    def body(*refs):
        if has_res:
            (oe, e8, bgat, n1s, n1b, w1, b1, w2, b2, n2s, n2b,
             xin, o_ref) = refs
        else:
            (oe, e8, bgat, n1s, n1b, w1, b1, w2, b2, n2s, n2b,
             o_ref) = refs
        g = oe[:, 0:HID]
        r8 = 1.0 / (oe[:, HID:HID + 8] + 1e-16)
        rexp = jnp.dot(r8, e8[...], preferred_element_type=_F32,
                       precision=_PREC)
        g = g * rexp + bgat[...]
        if has_res:
            g = g + xin[...]
        mu = jnp.mean(g, axis=-1, keepdims=True)
        dgl = g - mu
        var = jnp.mean(dgl * dgl, axis=-1, keepdims=True)
        x1 = dgl * lax.rsqrt(var + 1e-5) * n1s[...] + n1b[...]
        f = jnp.dot(x1, w1[...], preferred_element_type=_F32,
                    precision=_PREC) + b1[...]
        f = jnp.maximum(f, 0.0)
        f = jnp.dot(f, w2[...], preferred_element_type=_F32,
                    precision=_PREC) + b2[...]
        x2 = x1 + f
        mu2 = jnp.mean(x2, axis=-1, keepdims=True)
        d2 = x2 - mu2
        var2 = jnp.mean(d2 * d2, axis=-1, keepdims=True)
        o_ref[...] = d2 * lax.rsqrt(var2 + 1e-5) * n2s[...] + n2b[...]

    return body


def _ffn(oe, e8, bgat, n1s, n1b, w1, b1, w2, b2, n2s, n2b, xin, rows_blk):
    ni = NN // rows_blk
    has_res = xin is not None
    in_specs = [
        pl.BlockSpec((rows_blk, MROW), lambda i: (i, 0)),
        pl.BlockSpec((8, HID), lambda i: (0, 0)),
        pl.BlockSpec((1, HID), lambda i: (0, 0)),
        pl.BlockSpec((1, HID), lambda i: (0, 0)),
        pl.BlockSpec((1, HID), lambda i: (0, 0)),
        pl.BlockSpec((HID, 2 * HID), lambda i: (0, 0)),
        pl.BlockSpec((1, 2 * HID), lambda i: (0, 0)),
        pl.BlockSpec((2 * HID, HID), lambda i: (0, 0)),
        pl.BlockSpec((1, HID), lambda i: (0, 0)),
        pl.BlockSpec((1, HID), lambda i: (0, 0)),
        pl.BlockSpec((1, HID), lambda i: (0, 0)),
    ]
    args = [oe, e8, bgat, n1s, n1b, w1, b1, w2, b2, n2s, n2b]
    if has_res:
        in_specs.append(pl.BlockSpec((rows_blk, HID), lambda i: (i, 0)))
        args.append(xin)
    return pl.pallas_call(
        _make_ffn_body(has_res),
        grid=(ni,),
        in_specs=in_specs,
        out_specs=pl.BlockSpec((rows_blk, HID), lambda i: (i, 0)),
        out_shape=jax.ShapeDtypeStruct((NN, HID), _F32),
    )(*args)


def kernel(x, edge_index, params):
    src = edge_index[0]
    dst = edge_index[1]
    dst2d = dst.reshape(1, EE)
    eye8 = jnp.eye(8, dtype=_F32)
    e8 = jnp.kron(eye8, jnp.ones((1, 32), _F32))

    for li, p in enumerate(params):
        a_s = (p["a_src"][:, :, None] * eye8[:, None, :]).reshape(HID, 8)
        a_d = (p["a_dst"][:, :, None] * eye8[:, None, :]).reshape(HID, 8)
        was16 = jnp.pad(jnp.tile(p["W_gat"] @ a_s, (1, 2)),
                        ((0, 0), (0, HALF - 16)))
        wad16 = jnp.pad(jnp.tile(p["W_gat"] @ a_d, (1, 2)),
                        ((0, 0), (0, HALF - 16)))
        h, asP, adP = _proj(x, p["W_gat"], was16, wad16, 1000)
        msg = _edges(src, dst, asP, adP, h)
        if isinstance(msg, (list, tuple)):
            msg = msg[0]
        oe = _reduce(dst2d, msg)
        xin = x if x.shape[-1] == HID else None
        x = _ffn(oe, e8,
                 p["b_gat"].reshape(1, HID),
                 p["n1_s"].reshape(1, HID), p["n1_b"].reshape(1, HID),
                 p["W1"], p["b1"].reshape(1, 2 * HID),
                 p["W2"], p["b2"].reshape(1, HID),
                 p["n2_s"].reshape(1, HID), p["n2_b"].reshape(1, HID),
                 xin, 1000)
    return x
